# baseline (speedup 1.0000x reference)
import jax
import jax.numpy as jnp
from jax.experimental import pallas as pl
from jax.experimental.pallas import tpu as pltpu

_NUM_STATES = 1024
_VOCAB = 100000
_BATCH = 8
_BN = 6272
_NB = (_VOCAB + _BN - 1) // _BN
_VPAD = _NB * _BN


def _hmm_obs_kernel(z_ref, wt_ref, b_ref, o_ref, m_ref, s_ref):
    j = pl.program_id(0)

    @pl.when(j == 0)
    def _init():
        m_ref[...] = jnp.full_like(m_ref, -jnp.inf)
        s_ref[...] = jnp.zeros_like(s_ref)

    x = jax.lax.dot_general(
        z_ref[...], wt_ref[...],
        dimension_numbers=(((1,), (1,)), ((), ())),
        preferred_element_type=jnp.float32) + b_ref[:, pl.ds(j * _BN, _BN)]

    @pl.when(j < _NB - 1)
    def _store_full():
        o_ref[:, pl.ds(j * _BN, _BN)] = x

    @pl.when(j == _NB - 1)
    def _store_tail():
        o_ref[:, (_NB - 1) * _BN:_VOCAB] = x[:, :_VOCAB - (_NB - 1) * _BN]

    col = j * _BN + jax.lax.broadcasted_iota(jnp.int32, (1, _BN), 1)
    xm = jnp.where(col < _VOCAB, x, -jnp.inf)

    m_old = m_ref[:, :1]
    s_old = s_ref[:, :1]
    bm = jnp.max(xm, axis=-1, keepdims=True)
    m_new = jnp.maximum(m_old, bm)
    s_new = s_old * jnp.exp(m_old - m_new) + jnp.sum(
        jnp.exp(xm - m_new), axis=-1, keepdims=True)
    m_ref[...] = jnp.broadcast_to(m_new, m_ref.shape)
    s_ref[...] = jnp.broadcast_to(s_new, s_ref.shape)

    @pl.when(j == _NB - 1)
    def _normalize():
        lse = m_new + jnp.log(s_new)
        o_ref[...] = o_ref[...] - lse


def kernel(z, W_obs, b_obs):
    b2d = jnp.pad(b_obs, (0, _VPAD - _VOCAB)).reshape(1, _VPAD)
    wt = jnp.swapaxes(W_obs, 0, 1)
    out = pl.pallas_call(
        _hmm_obs_kernel,
        grid=(_NB,),
        in_specs=[
            pl.BlockSpec((_BATCH, _NUM_STATES), lambda j: (0, 0)),
            pl.BlockSpec((_BN, _NUM_STATES), lambda j: (j, 0)),
            pl.BlockSpec((1, _VPAD), lambda j: (0, 0)),
        ],
        out_specs=pl.BlockSpec((_BATCH, _VOCAB), lambda j: (0, 0)),
        out_shape=jax.ShapeDtypeStruct((_BATCH, _VOCAB), jnp.float32),
        scratch_shapes=[
            pltpu.VMEM((_BATCH, 128), jnp.float32),
            pltpu.VMEM((_BATCH, 128), jnp.float32),
        ],
        compiler_params=pltpu.CompilerParams(
            dimension_semantics=("arbitrary",),
        ),
    )(z, wt, b2d)
    return out


# speedup vs baseline: 1.0022x; 1.0022x over previous
import jax
import jax.numpy as jnp
from jax.experimental import pallas as pl
from jax.experimental.pallas import tpu as pltpu

_NUM_STATES = 1024
_VOCAB = 100000
_BATCH = 8
_BN = 4096
_NBM = 24
_TOFF = _NBM * _BN
_TBN = 2048
_NB = _NBM + 1
_BPAD = _TOFF + _TBN

_DN = (((1,), (1,)), ((), ()))


def _hmm_obs_kernel(z_ref, wt_ref, wtt_ref, b_ref, o_ref, m_ref, s_ref):
    j = pl.program_id(0)

    @pl.when(j == 0)
    def _init():
        m_ref[...] = jnp.full_like(m_ref, -jnp.inf)
        s_ref[...] = jnp.zeros_like(s_ref)

    @pl.when(j < _NBM)
    def _main():
        x = jax.lax.dot_general(
            z_ref[...], wt_ref[...], dimension_numbers=_DN,
            preferred_element_type=jnp.float32) + b_ref[:, pl.ds(j * _BN, _BN)]
        o_ref[:, pl.ds(j * _BN, _BN)] = x
        m_old = m_ref[:, :1]
        s_old = s_ref[:, :1]
        bm = jnp.max(x, axis=-1, keepdims=True)
        m_new = jnp.maximum(m_old, bm)
        s_new = s_old * jnp.exp(m_old - m_new) + jnp.sum(
            jnp.exp(x - m_new), axis=-1, keepdims=True)
        m_ref[...] = jnp.broadcast_to(m_new, m_ref.shape)
        s_ref[...] = jnp.broadcast_to(s_new, s_ref.shape)

    @pl.when(j == _NBM)
    def _tail():
        x = jax.lax.dot_general(
            z_ref[...], wtt_ref[...], dimension_numbers=_DN,
            preferred_element_type=jnp.float32) + b_ref[:, pl.ds(_TOFF, _TBN)]
        o_ref[:, _TOFF:_VOCAB] = x[:, :_VOCAB - _TOFF]
        col = _TOFF + jax.lax.broadcasted_iota(jnp.int32, (1, _TBN), 1)
        xm = jnp.where(col < _VOCAB, x, -jnp.inf)
        m_old = m_ref[:, :1]
        s_old = s_ref[:, :1]
        bm = jnp.max(xm, axis=-1, keepdims=True)
        m_new = jnp.maximum(m_old, bm)
        s_new = s_old * jnp.exp(m_old - m_new) + jnp.sum(
            jnp.exp(xm - m_new), axis=-1, keepdims=True)
        lse = m_new + jnp.log(s_new)
        o_ref[...] = o_ref[...] - lse


def kernel(z, W_obs, b_obs):
    b2d = jnp.pad(b_obs, (0, _BPAD - _VOCAB)).reshape(1, _BPAD)
    wt = jnp.swapaxes(W_obs, 0, 1)
    out = pl.pallas_call(
        _hmm_obs_kernel,
        grid=(_NB,),
        in_specs=[
            pl.BlockSpec((_BATCH, _NUM_STATES), lambda j: (0, 0)),
            pl.BlockSpec((_BN, _NUM_STATES),
                         lambda j: (jnp.minimum(j, _NBM - 1), 0)),
            pl.BlockSpec((_TBN, _NUM_STATES), lambda j: (_TOFF // _TBN, 0)),
            pl.BlockSpec((1, _BPAD), lambda j: (0, 0)),
        ],
        out_specs=pl.BlockSpec((_BATCH, _VOCAB), lambda j: (0, 0)),
        out_shape=jax.ShapeDtypeStruct((_BATCH, _VOCAB), jnp.float32),
        scratch_shapes=[
            pltpu.VMEM((_BATCH, 128), jnp.float32),
            pltpu.VMEM((_BATCH, 128), jnp.float32),
        ],
        compiler_params=pltpu.CompilerParams(
            dimension_semantics=("arbitrary",),
        ),
    )(z, wt, wt, b2d)
    return out


# speedup vs baseline: 1.0080x; 1.0057x over previous
import jax
import jax.numpy as jnp
from jax.experimental import pallas as pl
from jax.experimental.pallas import tpu as pltpu

_NUM_STATES = 1024
_VOCAB = 100000
_BATCH = 8
_BN = 4096
_NB = (_VOCAB + _BN - 1) // _BN
_VPAD = _NB * _BN


def _hmm_obs_kernel(z_ref, wt_ref, b_ref, o_ref, m_ref, s_ref):
    j = pl.program_id(0)

    @pl.when(j == 0)
    def _init():
        m_ref[...] = jnp.full_like(m_ref, -jnp.inf)
        s_ref[...] = jnp.zeros_like(s_ref)

    x = jax.lax.dot_general(
        z_ref[...], wt_ref[...],
        dimension_numbers=(((1,), (1,)), ((), ())),
        preferred_element_type=jnp.float32) + b_ref[:, pl.ds(j * _BN, _BN)]

    @pl.when(j < _NB - 1)
    def _store_full():
        o_ref[:, pl.ds(j * _BN, _BN)] = x

    @pl.when(j == _NB - 1)
    def _store_tail():
        o_ref[:, (_NB - 1) * _BN:_VOCAB] = x[:, :_VOCAB - (_NB - 1) * _BN]

    col = j * _BN + jax.lax.broadcasted_iota(jnp.int32, (1, _BN), 1)
    xm = jnp.where(col < _VOCAB, x, -jnp.inf)

    m_old = m_ref[:, :1]
    s_old = s_ref[:, :1]
    bm = jnp.max(xm, axis=-1, keepdims=True)
    m_new = jnp.maximum(m_old, bm)
    s_new = s_old * jnp.exp(m_old - m_new) + jnp.sum(
        jnp.exp(xm - m_new), axis=-1, keepdims=True)
    m_ref[...] = jnp.broadcast_to(m_new, m_ref.shape)
    s_ref[...] = jnp.broadcast_to(s_new, s_ref.shape)

    @pl.when(j == _NB - 1)
    def _normalize():
        lse = m_new + jnp.log(s_new)
        o_ref[...] = o_ref[...] - lse


def kernel(z, W_obs, b_obs):
    b2d = jnp.pad(b_obs, (0, _VPAD - _VOCAB)).reshape(1, _VPAD)
    wt = jnp.swapaxes(W_obs, 0, 1)
    out = pl.pallas_call(
        _hmm_obs_kernel,
        grid=(_NB,),
        in_specs=[
            pl.BlockSpec((_BATCH, _NUM_STATES), lambda j: (0, 0)),
            pl.BlockSpec((_BN, _NUM_STATES), lambda j: (j, 0)),
            pl.BlockSpec((1, _VPAD), lambda j: (0, 0)),
        ],
        out_specs=pl.BlockSpec((_BATCH, _VOCAB), lambda j: (0, 0)),
        out_shape=jax.ShapeDtypeStruct((_BATCH, _VOCAB), jnp.float32),
        scratch_shapes=[
            pltpu.VMEM((_BATCH, 128), jnp.float32),
            pltpu.VMEM((_BATCH, 128), jnp.float32),
        ],
        compiler_params=pltpu.CompilerParams(
            dimension_semantics=("arbitrary",),
        ),
    )(z, wt, b2d)
    return out
